# baseline (speedup 1.0000x reference)
import functools

import jax
import jax.numpy as jnp
from jax import lax
from jax.experimental import pallas as pl
from jax.experimental.pallas import tpu as pltpu
from jax.experimental.pallas import tpu_sc as plsc

VOCAB = 1000
BATCH = 1024
SEQ = 50
BL = BATCH * SEQ
NC = 2
NS = 16
NW = NC * NS
ITEMS_PER_W = BATCH // NW
LOGZ_PAD = 1024


def _logz_body(table_ref, out_ref):
    x = table_ref[...]
    m = jnp.max(x, axis=1, keepdims=True)
    s = jnp.sum(jnp.exp(x - m), axis=1, keepdims=True)
    out_ref[...] = m + jnp.log(s)


def _compute_logz(table):
    return pl.pallas_call(
        _logz_body,
        out_shape=jax.ShapeDtypeStruct((VOCAB, 1), jnp.float32),
    )(table)


_mesh = plsc.VectorSubcoreMesh(core_axis_name="c", subcore_axis_name="s")


@functools.partial(
    pl.kernel,
    out_type=[
        jax.ShapeDtypeStruct((BATCH, SEQ, VOCAB), jnp.float32),
        jax.ShapeDtypeStruct((NW, 16), jnp.float32),
    ],
    mesh=_mesh,
    compiler_params=pltpu.CompilerParams(
        needs_layout_passes=False, use_tc_tiling_on_sc=False),
    scratch_types=[
        pltpu.VMEM((ITEMS_PER_W, SEQ), jnp.int32),
        pltpu.VMEM((ITEMS_PER_W, SEQ), jnp.int32),
        pltpu.VMEM((SEQ, VOCAB), jnp.float32),
        pltpu.VMEM((SEQ, VOCAB), jnp.float32),
        pltpu.VMEM((LOGZ_PAD,), jnp.float32),
        pltpu.VMEM((16,), jnp.float32),
        pltpu.SemaphoreType.DMA,
        pltpu.SemaphoreType.DMA,
        pltpu.SemaphoreType.DMA,
        pltpu.SemaphoreType.DMA,
    ],
)
def _sc_main(table_hbm, idx_hbm, tgt_hbm, logz_hbm,
             out_hbm, part_hbm,
             idx_v, tgt_v, rows0, rows1, logz_v, acc_v,
             gsem0, gsem1, wsem0, wsem1):
    wid = lax.axis_index("s") * NC + lax.axis_index("c")
    item0 = wid * ITEMS_PER_W
    rows = (rows0, rows1)
    gsems = (gsem0, gsem1)
    wsems = (wsem0, wsem1)

    pltpu.sync_copy(logz_hbm, logz_v)
    pltpu.sync_copy(idx_hbm.at[pl.ds(item0, ITEMS_PER_W)], idx_v)
    pltpu.sync_copy(tgt_hbm.at[pl.ds(item0, ITEMS_PER_W)], tgt_v)

    def start_gather(c, b):
        pltpu.async_copy(table_hbm.at[idx_v.at[c]], rows[b], gsems[b])

    def wait_gather(b):
        pltpu.make_async_copy(table_hbm.at[pl.ds(0, SEQ)],
                              rows[b], gsems[b]).wait()

    def wait_write(b):
        pltpu.make_async_copy(rows[b], out_hbm.at[0], wsems[b]).wait()

    def loss_chunk(c, b, acc):
        cvec = jnp.full((16,), c, jnp.int32)
        for j in range(0, SEQ, 16):
            nvalid = min(SEQ - j, 16)
            lanes = lax.iota(jnp.int32, 16)
            rowids = lanes + j
            if nvalid < 16:
                valid = lanes < nvalid
                rowids = jnp.where(valid, rowids, 0)
            ids = plsc.load_gather(idx_v, [cvec, rowids])
            tgts = plsc.load_gather(tgt_v, [cvec, rowids])
            lz = plsc.load_gather(logz_v, [ids])
            pk = plsc.load_gather(rows[b], [rowids, tgts])
            nll = lz - pk
            if nvalid < 16:
                nll = jnp.where(valid, nll, 0.0)
            acc = acc + nll
        return acc

    start_gather(0, 0)
    start_gather(1, 1)

    def body(c2, acc):
        for b in range(2):
            c = c2 * 2 + b
            wait_gather(b)
            pltpu.async_copy(rows[b], out_hbm.at[item0 + c], wsems[b])
            acc = loss_chunk(c, b, acc)

            @pl.when(c + 2 < ITEMS_PER_W)
            def _():
                wait_write(b)
                start_gather(c + 2, b)

        return acc

    acc = lax.fori_loop(0, ITEMS_PER_W // 2, body,
                        jnp.zeros((16,), jnp.float32))
    wait_write(0)
    wait_write(1)
    acc_v[...] = acc
    pltpu.sync_copy(acc_v, part_hbm.at[wid])


def _loss_body(part_ref, out_ref):
    total = jnp.sum(part_ref[...]) * (1.0 / BL)
    out_ref[...] = jnp.reshape(total, (1, 1))


def _finalize_loss(partials):
    return pl.pallas_call(
        _loss_body,
        out_shape=jax.ShapeDtypeStruct((1, 1), jnp.float32),
    )(partials)


def kernel(idx, targets, logits_table):
    idx_i = idx.astype(jnp.int32)
    tgt_i = targets.astype(jnp.int32)
    table = logits_table.astype(jnp.float32)
    logz = _compute_logz(table)
    logz_pad = jnp.pad(logz[:, 0], (0, LOGZ_PAD - VOCAB))
    logits, partials = _sc_main(table, idx_i, tgt_i, logz_pad)
    loss = _finalize_loss(partials)[0, 0]
    return logits, loss


# speedup vs baseline: 1.0106x; 1.0106x over previous
import functools

import jax
import jax.numpy as jnp
from jax import lax
from jax.experimental import pallas as pl
from jax.experimental.pallas import tpu as pltpu
from jax.experimental.pallas import tpu_sc as plsc

VOCAB = 1000
BATCH = 1024
SEQ = 50
BL = BATCH * SEQ
NC = 2
NS = 16
NW = NC * NS
NBG = BATCH // 128
NSBG = SEQ * NBG
KCH = 5
KW = VOCAB // KCH
KV8 = KW // 8
LOGZ_PAD = 1024


def _logz_body(table_ref, out_ref):
    x = table_ref[...]
    m = jnp.max(x, axis=1, keepdims=True)
    s = jnp.sum(jnp.exp(x - m), axis=1, keepdims=True)
    out_ref[...] = m + jnp.log(s)


def _compute_logz(table):
    return pl.pallas_call(
        _logz_body,
        out_shape=jax.ShapeDtypeStruct((VOCAB, 1), jnp.float32),
    )(table)


_mesh = plsc.VectorSubcoreMesh(core_axis_name="c", subcore_axis_name="s")


@functools.partial(
    pl.kernel,
    out_type=[
        jax.ShapeDtypeStruct((SEQ, VOCAB // 8, NBG, 8, 128), jnp.float32),
        jax.ShapeDtypeStruct((NW, 16), jnp.float32),
    ],
    mesh=_mesh,
    compiler_params=pltpu.CompilerParams(
        needs_layout_passes=False, use_tc_tiling_on_sc=False),
    scratch_types=[
        pltpu.VMEM((4, 128), jnp.int32),
        pltpu.VMEM((4, 128), jnp.int32),
        pltpu.VMEM((128, KW), jnp.float32),
        pltpu.VMEM((128, KW), jnp.float32),
        pltpu.VMEM((KV8, 8, 128), jnp.float32),
        pltpu.VMEM((KV8, 8, 128), jnp.float32),
        pltpu.VMEM((LOGZ_PAD,), jnp.float32),
        pltpu.VMEM((16,), jnp.float32),
        pltpu.SemaphoreType.DMA,
        pltpu.SemaphoreType.DMA,
        pltpu.SemaphoreType.DMA,
        pltpu.SemaphoreType.DMA,
        pltpu.SemaphoreType.DMA,
    ],
)
def _sc_main(t0_hbm, t1_hbm, t2_hbm, t3_hbm, t4_hbm,
             idxT_hbm, tgtT_hbm, logz_hbm,
             out_hbm, part_hbm,
             idxc, tgtc, g0, g1, tr0, tr1, logz_v, acc_v,
             gsem0, gsem1, wsem0, wsem1, isem):
    wid = lax.axis_index("s") * NC + lax.axis_index("c")
    tks = (t0_hbm, t1_hbm, t2_hbm, t3_hbm, t4_hbm)
    g = (g0, g1)
    tr = (tr0, tr1)
    gsems = (gsem0, gsem1)
    wsems = (wsem0, wsem1)
    lanes = lax.iota(jnp.int32, 16)

    sbg_lo = 12 * wid + 2 * jnp.minimum(wid, 8)
    n_sbg = jnp.where(wid < 8, 14, 12)
    nblk = n_sbg // 2

    pltpu.sync_copy(logz_hbm, logz_v)

    def fire_cols(sbg):
        s = sbg // NBG
        bg = lax.rem(sbg, NBG)
        slot = lax.rem(sbg, 4)
        pltpu.async_copy(idxT_hbm.at[s, pl.ds(bg * 128, 128)],
                         idxc.at[slot], isem)
        pltpu.async_copy(tgtT_hbm.at[s, pl.ds(bg * 128, 128)],
                         tgtc.at[slot], isem)

    def wait_cols_pair():
        pltpu.make_async_copy(idxT_hbm.at[0, pl.ds(0, 128)],
                              idxc.at[0], isem).wait()
        pltpu.make_async_copy(tgtT_hbm.at[0, pl.ds(0, 128)],
                              tgtc.at[0], isem).wait()

    def fire_gather(sbg, k, b):
        slot = lax.rem(sbg, 4)
        pltpu.async_copy(tks[k].at[idxc.at[slot]], g[b], gsems[b])

    def wait_gather(b):
        pltpu.make_async_copy(t0_hbm.at[pl.ds(0, 128)], g[b],
                              gsems[b]).wait()

    def fire_write(sbg, k, b):
        s = sbg // NBG
        bg = lax.rem(sbg, NBG)
        pltpu.async_copy(tr[b], out_hbm.at[s, pl.ds(k * KV8, KV8), bg],
                         wsems[b])

    def wait_write(b):
        pltpu.make_async_copy(tr[b], out_hbm.at[0, pl.ds(0, KV8), 0],
                              wsems[b]).wait()

    def loss_item(sbg, k, b, acc):
        slot = lax.rem(sbg, 4)
        for j in range(8):
            tgts = tgtc[slot, pl.ds(j * 16, 16)]
            local = tgts - (k * KW)
            valid = (local >= 0) & (local < KW)
            lsafe = jnp.where(valid, local, 0)
            pk = plsc.load_gather(g[b], [lanes + (j * 16), lsafe])
            acc = acc - jnp.where(valid, pk, 0.0)
            if k == 0:
                ids = idxc[slot, pl.ds(j * 16, 16)]
                acc = acc + plsc.load_gather(logz_v, [ids])
        return acc

    def transpose_item(b):
        rowvecs = [lanes + (j * 16) for j in range(8)]

        def body(v8, carry):
            for v0 in range(8):
                cvec = jnp.full((16,), v8 * 8 + v0, jnp.int32)
                for j in range(8):
                    x = plsc.load_gather(g[b], [rowvecs[j], cvec])
                    tr[b][v8, v0, pl.ds(j * 16, 16)] = x
            return carry

        lax.fori_loop(0, KV8, body, 0)

    fire_cols(sbg_lo)
    fire_cols(sbg_lo + 1)
    wait_cols_pair()
    wait_cols_pair()

    def block(blk, acc):
        sbg_a = sbg_lo + blk * 2

        @pl.when(blk > 0)
        def _():
            wait_cols_pair()
            wait_cols_pair()

        @pl.when(sbg_a + 2 < sbg_lo + n_sbg)
        def _():
            fire_cols(sbg_a + 2)

        @pl.when(sbg_a + 3 < sbg_lo + n_sbg)
        def _():
            fire_cols(sbg_a + 3)

        fire_gather(sbg_a, 0, 0)
        fire_gather(sbg_a, 1, 1)

        for m in range(10):
            i = blk * 10 + m
            k = m % 5
            b = m % 2
            sbg = sbg_a + (m // 5)
            wait_gather(b)
            acc = loss_item(sbg, k, b, acc)

            @pl.when(i >= 2)
            def _():
                wait_write(b)

            transpose_item(b)
            fire_write(sbg, k, b)

            if m <= 7:
                fire_gather(sbg_a + ((m + 2) // 5), (m + 2) % 5, b)

        return acc

    acc = lax.fori_loop(0, nblk, block, jnp.zeros((16,), jnp.float32))
    wait_write(0)
    wait_write(1)
    acc_v[...] = acc
    pltpu.sync_copy(acc_v, part_hbm.at[wid])


def _loss_body(part_ref, out_ref):
    total = jnp.sum(part_ref[...]) * (1.0 / BL)
    out_ref[...] = jnp.reshape(total, (1, 1))


def _finalize_loss(partials):
    return pl.pallas_call(
        _loss_body,
        out_shape=jax.ShapeDtypeStruct((1, 1), jnp.float32),
    )(partials)


def kernel(idx, targets, logits_table):
    idxT = idx.astype(jnp.int32).T
    tgtT = targets.astype(jnp.int32).T
    table = logits_table.astype(jnp.float32)
    logz = _compute_logz(table)
    logz_pad = jnp.pad(logz[:, 0], (0, LOGZ_PAD - VOCAB))
    tslices = [table[:, k * KW:(k + 1) * KW] for k in range(KCH)]
    q, partials = _sc_main(*tslices, idxT, tgtT, logz_pad)
    logits = jnp.transpose(q, (2, 4, 0, 1, 3)).reshape(BATCH, SEQ, VOCAB)
    loss = _finalize_loss(partials)[0, 0]
    return logits, loss


# speedup vs baseline: 3.1868x; 3.1533x over previous
import functools

import jax
import jax.numpy as jnp
from jax import lax
from jax.experimental import pallas as pl
from jax.experimental.pallas import tpu as pltpu
from jax.experimental.pallas import tpu_sc as plsc

VOCAB = 1000
BATCH = 1024
SEQ = 50
BL = BATCH * SEQ
NC = 2
NS = 16
NW = NC * NS
NBG = BATCH // 128
NSBG = SEQ * NBG
KCH = 5
KW = VOCAB // KCH
KV8 = KW // 8
LOGZ_PAD = 1024


def _logz_body(table_ref, out_ref):
    x = table_ref[...]
    m = jnp.max(x, axis=1, keepdims=True)
    s = jnp.sum(jnp.exp(x - m), axis=1, keepdims=True)
    out_ref[...] = m + jnp.log(s)


def _compute_logz(table):
    return pl.pallas_call(
        _logz_body,
        out_shape=jax.ShapeDtypeStruct((VOCAB, 1), jnp.float32),
    )(table)


_mesh = plsc.VectorSubcoreMesh(core_axis_name="c", subcore_axis_name="s")


@functools.partial(
    pl.kernel,
    out_type=[
        jax.ShapeDtypeStruct((SEQ, VOCAB // 8, NBG, 8, 128), jnp.float32),
        jax.ShapeDtypeStruct((NW, 16), jnp.float32),
    ],
    mesh=_mesh,
    compiler_params=pltpu.CompilerParams(
        needs_layout_passes=False, use_tc_tiling_on_sc=False),
    scratch_types=[
        pltpu.VMEM((4, 128), jnp.int32),
        pltpu.VMEM((4, 128), jnp.int32),
        pltpu.VMEM((128, KW), jnp.float32),
        pltpu.VMEM((128, KW), jnp.float32),
        pltpu.VMEM((KV8, 8, 128), jnp.float32),
        pltpu.VMEM((KV8, 8, 128), jnp.float32),
        pltpu.VMEM((LOGZ_PAD,), jnp.float32),
        pltpu.VMEM((16,), jnp.float32),
        pltpu.SemaphoreType.DMA,
        pltpu.SemaphoreType.DMA,
        pltpu.SemaphoreType.DMA,
        pltpu.SemaphoreType.DMA,
        pltpu.SemaphoreType.DMA,
    ],
)
def _sc_main(t0_hbm, t1_hbm, t2_hbm, t3_hbm, t4_hbm,
             idxT_hbm, tgtT_hbm, logz_hbm,
             out_hbm, part_hbm,
             idxc, tgtc, g0, g1, tr0, tr1, logz_v, acc_v,
             gsem0, gsem1, wsem0, wsem1, isem):
    wid = lax.axis_index("s") * NC + lax.axis_index("c")
    tks = (t0_hbm, t1_hbm, t2_hbm, t3_hbm, t4_hbm)
    g = (g0, g1)
    tr = (tr0, tr1)
    gsems = (gsem0, gsem1)
    wsems = (wsem0, wsem1)
    lanes = lax.iota(jnp.int32, 16)

    sbg_lo = 12 * wid + 2 * jnp.minimum(wid, 8)
    n_sbg = jnp.where(wid < 8, 14, 12)
    nblk = n_sbg // 2

    pltpu.sync_copy(logz_hbm, logz_v)

    def fire_cols(sbg):
        s = sbg // NBG
        bg = lax.rem(sbg, NBG)
        slot = lax.rem(sbg, 4)
        pltpu.async_copy(idxT_hbm.at[s, pl.ds(bg * 128, 128)],
                         idxc.at[slot], isem)
        pltpu.async_copy(tgtT_hbm.at[s, pl.ds(bg * 128, 128)],
                         tgtc.at[slot], isem)

    def wait_cols_pair():
        pltpu.make_async_copy(idxT_hbm.at[0, pl.ds(0, 128)],
                              idxc.at[0], isem).wait()
        pltpu.make_async_copy(tgtT_hbm.at[0, pl.ds(0, 128)],
                              tgtc.at[0], isem).wait()

    def fire_gather(sbg, k, b):
        slot = lax.rem(sbg, 4)
        pltpu.async_copy(tks[k].at[idxc.at[slot]], g[b], gsems[b])

    def wait_gather(b):
        pltpu.make_async_copy(t0_hbm.at[pl.ds(0, 128)], g[b],
                              gsems[b]).wait()

    def fire_write(sbg, k, b):
        s = sbg // NBG
        bg = lax.rem(sbg, NBG)
        pltpu.async_copy(tr[b], out_hbm.at[s, pl.ds(k * KV8, KV8), bg],
                         wsems[b])

    def wait_write(b):
        pltpu.make_async_copy(tr[b], out_hbm.at[0, pl.ds(0, KV8), 0],
                              wsems[b]).wait()

    def loss_item(sbg, k, b, acc):
        slot = lax.rem(sbg, 4)
        for j in range(8):
            tgts = tgtc[slot, pl.ds(j * 16, 16)]
            local = tgts - (k * KW)
            valid = (local >= 0) & (local < KW)
            lsafe = jnp.where(valid, local, 0)
            pk = plsc.load_gather(g[b], [lanes + (j * 16), lsafe])
            acc = acc - jnp.where(valid, pk, 0.0)
            if k == 0:
                ids = idxc[slot, pl.ds(j * 16, 16)]
                acc = acc + plsc.load_gather(logz_v, [ids])
        return acc

    def transpose_item(b):
        rowvecs = [lanes + (j * 16) for j in range(8)]

        @plsc.parallel_loop(0, KW, step=1, unroll=1)
        def _(c):
            v8 = c // 8
            v0 = lax.rem(c, 8)
            cvec = jnp.full((16,), c, jnp.int32)
            for j in range(8):
                x = plsc.load_gather(g[b], [rowvecs[j], cvec])
                tr[b][v8, v0, pl.ds(j * 16, 16)] = x

    fire_cols(sbg_lo)
    fire_cols(sbg_lo + 1)
    wait_cols_pair()
    wait_cols_pair()

    def block(blk, acc):
        sbg_a = sbg_lo + blk * 2

        @pl.when(blk > 0)
        def _():
            wait_cols_pair()
            wait_cols_pair()

        @pl.when(sbg_a + 2 < sbg_lo + n_sbg)
        def _():
            fire_cols(sbg_a + 2)

        @pl.when(sbg_a + 3 < sbg_lo + n_sbg)
        def _():
            fire_cols(sbg_a + 3)

        fire_gather(sbg_a, 0, 0)
        fire_gather(sbg_a, 1, 1)

        for m in range(10):
            i = blk * 10 + m
            k = m % 5
            b = m % 2
            sbg = sbg_a + (m // 5)
            wait_gather(b)
            acc = loss_item(sbg, k, b, acc)

            @pl.when(i >= 2)
            def _():
                wait_write(b)

            transpose_item(b)
            fire_write(sbg, k, b)

            if m <= 7:
                fire_gather(sbg_a + ((m + 2) // 5), (m + 2) % 5, b)

        return acc

    acc = lax.fori_loop(0, nblk, block, jnp.zeros((16,), jnp.float32))
    wait_write(0)
    wait_write(1)
    acc_v[...] = acc
    pltpu.sync_copy(acc_v, part_hbm.at[wid])


def _loss_body(part_ref, out_ref):
    total = jnp.sum(part_ref[...]) * (1.0 / BL)
    out_ref[...] = jnp.reshape(total, (1, 1))


def _finalize_loss(partials):
    return pl.pallas_call(
        _loss_body,
        out_shape=jax.ShapeDtypeStruct((1, 1), jnp.float32),
    )(partials)


def kernel(idx, targets, logits_table):
    idxT = idx.astype(jnp.int32).T
    tgtT = targets.astype(jnp.int32).T
    table = logits_table.astype(jnp.float32)
    logz = _compute_logz(table)
    logz_pad = jnp.pad(logz[:, 0], (0, LOGZ_PAD - VOCAB))
    tslices = [table[:, k * KW:(k + 1) * KW] for k in range(KCH)]
    q, partials = _sc_main(*tslices, idxT, tgtT, logz_pad)
    logits = jnp.transpose(q, (2, 4, 0, 1, 3)).reshape(BATCH, SEQ, VOCAB)
    loss = _finalize_loss(partials)[0, 0]
    return logits, loss


# speedup vs baseline: 3.1876x; 1.0002x over previous
import functools

import jax
import jax.numpy as jnp
from jax import lax
from jax.experimental import pallas as pl
from jax.experimental.pallas import tpu as pltpu
from jax.experimental.pallas import tpu_sc as plsc

VOCAB = 1000
BATCH = 1024
SEQ = 50
BL = BATCH * SEQ
NC = 2
NS = 16
NW = NC * NS
NBG = BATCH // 128
NSBG = SEQ * NBG
KCH = 5
KW = VOCAB // KCH
KV8 = KW // 8
LOGZ_PAD = 1024


def _logz_body(table_ref, out_ref):
    x = table_ref[...]
    m = jnp.max(x, axis=1, keepdims=True)
    s = jnp.sum(jnp.exp(x - m), axis=1, keepdims=True)
    out_ref[...] = m + jnp.log(s)


def _compute_logz(table):
    return pl.pallas_call(
        _logz_body,
        out_shape=jax.ShapeDtypeStruct((VOCAB, 1), jnp.float32),
    )(table)


_mesh = plsc.VectorSubcoreMesh(core_axis_name="c", subcore_axis_name="s")


@functools.partial(
    pl.kernel,
    out_type=[
        jax.ShapeDtypeStruct((SEQ, VOCAB // 8, NBG, 8, 128), jnp.float32),
        jax.ShapeDtypeStruct((NW, 16), jnp.float32),
    ],
    mesh=_mesh,
    compiler_params=pltpu.CompilerParams(
        needs_layout_passes=False, use_tc_tiling_on_sc=False),
    scratch_types=[
        pltpu.VMEM((4, 128), jnp.int32),
        pltpu.VMEM((4, 128), jnp.int32),
        pltpu.VMEM((128, KW), jnp.float32),
        pltpu.VMEM((128, KW), jnp.float32),
        pltpu.VMEM((KV8, 8, 128), jnp.float32),
        pltpu.VMEM((KV8, 8, 128), jnp.float32),
        pltpu.VMEM((LOGZ_PAD,), jnp.float32),
        pltpu.VMEM((16,), jnp.float32),
        pltpu.SemaphoreType.DMA,
        pltpu.SemaphoreType.DMA,
        pltpu.SemaphoreType.DMA,
        pltpu.SemaphoreType.DMA,
        pltpu.SemaphoreType.DMA,
    ],
)
def _sc_main(t0_hbm, t1_hbm, t2_hbm, t3_hbm, t4_hbm,
             idxT_hbm, tgtT_hbm, logz_hbm,
             out_hbm, part_hbm,
             idxc, tgtc, g0, g1, tr0, tr1, logz_v, acc_v,
             gsem0, gsem1, wsem0, wsem1, isem):
    wid = lax.axis_index("s") * NC + lax.axis_index("c")
    tks = (t0_hbm, t1_hbm, t2_hbm, t3_hbm, t4_hbm)
    g = (g0, g1)
    tr = (tr0, tr1)
    gsems = (gsem0, gsem1)
    wsems = (wsem0, wsem1)
    lanes = lax.iota(jnp.int32, 16)

    sbg_lo = 12 * wid + 2 * jnp.minimum(wid, 8)
    n_sbg = jnp.where(wid < 8, 14, 12)
    nblk = n_sbg // 2

    pltpu.sync_copy(logz_hbm, logz_v)

    def fire_cols(sbg):
        s = sbg // NBG
        bg = lax.rem(sbg, NBG)
        slot = lax.rem(sbg, 4)
        pltpu.async_copy(idxT_hbm.at[s, pl.ds(bg * 128, 128)],
                         idxc.at[slot], isem)
        pltpu.async_copy(tgtT_hbm.at[s, pl.ds(bg * 128, 128)],
                         tgtc.at[slot], isem)

    def wait_cols_pair():
        pltpu.make_async_copy(idxT_hbm.at[0, pl.ds(0, 128)],
                              idxc.at[0], isem).wait()
        pltpu.make_async_copy(tgtT_hbm.at[0, pl.ds(0, 128)],
                              tgtc.at[0], isem).wait()

    def fire_gather(sbg, k, b):
        slot = lax.rem(sbg, 4)
        pltpu.async_copy(tks[k].at[idxc.at[slot]], g[b], gsems[b])

    def wait_gather(b):
        pltpu.make_async_copy(tks[0].at[pl.ds(0, 128)], g[b],
                              gsems[b]).wait()

    def fire_write(sbg, k, b):
        s = sbg // NBG
        bg = lax.rem(sbg, NBG)
        pltpu.async_copy(tr[b], out_hbm.at[s, pl.ds(k * KV8, KV8), bg],
                         wsems[b])

    def wait_write(b):
        pltpu.make_async_copy(tr[b], out_hbm.at[0, pl.ds(0, KV8), 0],
                              wsems[b]).wait()

    def loss_item(sbg, k, b, acc):
        slot = lax.rem(sbg, 4)
        for j in range(8):
            tgts = tgtc[slot, pl.ds(j * 16, 16)]
            local = tgts - (k * KW)
            valid = (local >= 0) & (local < KW)
            lsafe = jnp.where(valid, local, 0)
            pk = plsc.load_gather(g[b], [lanes + (j * 16), lsafe])
            acc = acc - jnp.where(valid, pk, 0.0)
            if k == 0:
                ids = idxc[slot, pl.ds(j * 16, 16)]
                acc = acc + plsc.load_gather(logz_v, [ids])
        return acc

    def transpose_item(b):
        rowvecs = [lanes + (j * 16) for j in range(8)]

        @plsc.parallel_loop(0, KW, step=1, unroll=2)
        def _(c):
            v8 = c // 8
            v0 = lax.rem(c, 8)
            cvec = jnp.full((16,), c, jnp.int32)
            for j in range(8):
                x = plsc.load_gather(g[b], [rowvecs[j], cvec])
                tr[b][v8, v0, pl.ds(j * 16, 16)] = x

    fire_cols(sbg_lo)
    fire_cols(sbg_lo + 1)
    wait_cols_pair()
    wait_cols_pair()

    def block(blk, acc):
        sbg_a = sbg_lo + blk * 2

        @pl.when(blk > 0)
        def _():
            wait_cols_pair()
            wait_cols_pair()

        @pl.when(sbg_a + 2 < sbg_lo + n_sbg)
        def _():
            fire_cols(sbg_a + 2)

        @pl.when(sbg_a + 3 < sbg_lo + n_sbg)
        def _():
            fire_cols(sbg_a + 3)

        fire_gather(sbg_a, 0, 0)
        fire_gather(sbg_a, 1, 1)

        for m in range(10):
            i = blk * 10 + m
            k = m % 5
            b = m % 2
            sbg = sbg_a + (m // 5)
            wait_gather(b)
            acc = loss_item(sbg, k, b, acc)

            @pl.when(i >= 2)
            def _():
                wait_write(b)

            transpose_item(b)
            fire_write(sbg, k, b)

            if m <= 7:
                fire_gather(sbg_a + ((m + 2) // 5), (m + 2) % 5, b)

        return acc

    acc = lax.fori_loop(0, nblk, block, jnp.zeros((16,), jnp.float32))
    wait_write(0)
    wait_write(1)
    acc_v[...] = acc
    pltpu.sync_copy(acc_v, part_hbm.at[wid])


def _loss_body(part_ref, out_ref):
    total = jnp.sum(part_ref[...]) * (1.0 / BL)
    out_ref[...] = jnp.reshape(total, (1, 1))


def _finalize_loss(partials):
    return pl.pallas_call(
        _loss_body,
        out_shape=jax.ShapeDtypeStruct((1, 1), jnp.float32),
    )(partials)


def kernel(idx, targets, logits_table):
    idxT = idx.astype(jnp.int32).T
    tgtT = targets.astype(jnp.int32).T
    table = logits_table.astype(jnp.float32)
    logz = _compute_logz(table)
    logz_pad = jnp.pad(logz[:, 0], (0, LOGZ_PAD - VOCAB))
    tslices = [table[:, k * KW:(k + 1) * KW] for k in range(KCH)]
    q, partials = _sc_main(*tslices, idxT, tgtT, logz_pad)
    logits = jnp.transpose(q, (2, 4, 0, 1, 3)).reshape(BATCH, SEQ, VOCAB)
    loss = _finalize_loss(partials)[0, 0]
    return logits, loss


# speedup vs baseline: 3.1880x; 1.0001x over previous
import functools

import jax
import jax.numpy as jnp
from jax import lax
from jax.experimental import pallas as pl
from jax.experimental.pallas import tpu as pltpu
from jax.experimental.pallas import tpu_sc as plsc

VOCAB = 1000
BATCH = 1024
SEQ = 50
BL = BATCH * SEQ
NC = 2
NS = 16
NW = NC * NS
NBG = BATCH // 128
NSBG = SEQ * NBG
KCH = 5
KW = VOCAB // KCH
KV8 = KW // 8
LOGZ_PAD = 1024


def _logz_body(table_ref, out_ref):
    x = table_ref[...]
    m = jnp.max(x, axis=1, keepdims=True)
    s = jnp.sum(jnp.exp(x - m), axis=1, keepdims=True)
    out_ref[...] = m + jnp.log(s)


def _compute_logz(table):
    return pl.pallas_call(
        _logz_body,
        out_shape=jax.ShapeDtypeStruct((VOCAB, 1), jnp.float32),
    )(table)


_mesh = plsc.VectorSubcoreMesh(core_axis_name="c", subcore_axis_name="s")


@functools.partial(
    pl.kernel,
    out_type=[
        jax.ShapeDtypeStruct((SEQ, VOCAB // 8, NBG, 8, 128), jnp.float32),
        jax.ShapeDtypeStruct((NW, 16), jnp.float32),
    ],
    mesh=_mesh,
    compiler_params=pltpu.CompilerParams(
        needs_layout_passes=False, use_tc_tiling_on_sc=False),
    scratch_types=[
        pltpu.VMEM((4, 128), jnp.int32),
        pltpu.VMEM((4, 128), jnp.int32),
        pltpu.VMEM((128, KW), jnp.float32),
        pltpu.VMEM((128, KW), jnp.float32),
        pltpu.VMEM((KV8, 8, 128), jnp.float32),
        pltpu.VMEM((KV8, 8, 128), jnp.float32),
        pltpu.VMEM((KV8, 8, 128), jnp.float32),
        pltpu.VMEM((LOGZ_PAD,), jnp.float32),
        pltpu.VMEM((16,), jnp.float32),
        pltpu.SemaphoreType.DMA,
        pltpu.SemaphoreType.DMA,
        pltpu.SemaphoreType.DMA,
        pltpu.SemaphoreType.DMA,
        pltpu.SemaphoreType.DMA,
        pltpu.SemaphoreType.DMA,
    ],
)
def _sc_main(t0_hbm, t1_hbm, t2_hbm, t3_hbm, t4_hbm,
             idxT_hbm, tgtT_hbm, logz_hbm,
             out_hbm, part_hbm,
             idxc, tgtc, g0, g1, tr0, tr1, tr2, logz_v, acc_v,
             gsem0, gsem1, wsem0, wsem1, wsem2, isem):
    wid = lax.axis_index("s") * NC + lax.axis_index("c")
    tks = (t0_hbm, t1_hbm, t2_hbm, t3_hbm, t4_hbm)
    g = (g0, g1)
    tr = (tr0, tr1, tr2)
    gsems = (gsem0, gsem1)
    wsems = (wsem0, wsem1, wsem2)
    lanes = lax.iota(jnp.int32, 16)

    sbg_lo = 12 * wid + 2 * jnp.minimum(wid, 8)
    n_sbg = jnp.where(wid < 8, 14, 12)
    nblk = n_sbg // 2

    pltpu.sync_copy(logz_hbm, logz_v)

    def fire_cols(sbg):
        s = sbg // NBG
        bg = lax.rem(sbg, NBG)
        slot = lax.rem(sbg, 4)
        pltpu.async_copy(idxT_hbm.at[s, pl.ds(bg * 128, 128)],
                         idxc.at[slot], isem)
        pltpu.async_copy(tgtT_hbm.at[s, pl.ds(bg * 128, 128)],
                         tgtc.at[slot], isem)

    def wait_cols_pair():
        pltpu.make_async_copy(idxT_hbm.at[0, pl.ds(0, 128)],
                              idxc.at[0], isem).wait()
        pltpu.make_async_copy(tgtT_hbm.at[0, pl.ds(0, 128)],
                              tgtc.at[0], isem).wait()

    def fire_gather(sbg, k, b):
        slot = lax.rem(sbg, 4)
        pltpu.async_copy(tks[k].at[idxc.at[slot]], g[b], gsems[b])

    def wait_gather(b):
        pltpu.make_async_copy(tks[0].at[pl.ds(0, 128)], g[b],
                              gsems[b]).wait()

    def fire_write(sbg, k, bt):
        s = sbg // NBG
        bg = lax.rem(sbg, NBG)
        pltpu.async_copy(tr[bt], out_hbm.at[s, pl.ds(k * KV8, KV8), bg],
                         wsems[bt])

    def wait_write(bt):
        pltpu.make_async_copy(tr[bt], out_hbm.at[0, pl.ds(0, KV8), 0],
                              wsems[bt]).wait()

    def loss_item(sbg, k, b, acc):
        slot = lax.rem(sbg, 4)
        for j in range(8):
            tgts = tgtc[slot, pl.ds(j * 16, 16)]
            local = tgts - (k * KW)
            valid = (local >= 0) & (local < KW)
            lsafe = jnp.where(valid, local, 0)
            pk = plsc.load_gather(g[b], [lanes + (j * 16), lsafe])
            acc = acc - jnp.where(valid, pk, 0.0)
            if k == 0:
                ids = idxc[slot, pl.ds(j * 16, 16)]
                acc = acc + plsc.load_gather(logz_v, [ids])
        return acc

    def transpose_item(b, bt):
        rowvecs = [lanes + (j * 16) for j in range(8)]

        @plsc.parallel_loop(0, KW, step=1, unroll=2)
        def _(c):
            v8 = c // 8
            v0 = lax.rem(c, 8)
            cvec = jnp.full((16,), c, jnp.int32)
            for j in range(8):
                x = plsc.load_gather(g[b], [rowvecs[j], cvec])
                tr[bt][v8, v0, pl.ds(j * 16, 16)] = x

    fire_cols(sbg_lo)
    fire_cols(sbg_lo + 1)
    wait_cols_pair()
    wait_cols_pair()

    def block(blk, acc):
        sbg_a = sbg_lo + blk * 2

        @pl.when(blk > 0)
        def _():
            wait_cols_pair()
            wait_cols_pair()

        @pl.when(sbg_a + 2 < sbg_lo + n_sbg)
        def _():
            fire_cols(sbg_a + 2)

        @pl.when(sbg_a + 3 < sbg_lo + n_sbg)
        def _():
            fire_cols(sbg_a + 3)

        fire_gather(sbg_a, 0, 0)
        fire_gather(sbg_a, 1, 1)

        for m in range(10):
            i = blk * 10 + m
            k = m % 5
            b = m % 2
            sbg = sbg_a + (m // 5)
            bt = m % 3
            wait_gather(b)
            acc = loss_item(sbg, k, b, acc)

            @pl.when(i >= 3)
            def _():
                wait_write(bt)

            transpose_item(b, bt)
            fire_write(sbg, k, bt)

            if m <= 7:
                fire_gather(sbg_a + ((m + 2) // 5), (m + 2) % 5, b)

        return acc

    acc = lax.fori_loop(0, nblk, block, jnp.zeros((16,), jnp.float32))
    wait_write(0)
    wait_write(1)
    wait_write(2)
    acc_v[...] = acc
    pltpu.sync_copy(acc_v, part_hbm.at[wid])


def _loss_body(part_ref, out_ref):
    total = jnp.sum(part_ref[...]) * (1.0 / BL)
    out_ref[...] = jnp.reshape(total, (1, 1))


def _finalize_loss(partials):
    return pl.pallas_call(
        _loss_body,
        out_shape=jax.ShapeDtypeStruct((1, 1), jnp.float32),
    )(partials)


def kernel(idx, targets, logits_table):
    idxT = idx.astype(jnp.int32).T
    tgtT = targets.astype(jnp.int32).T
    table = logits_table.astype(jnp.float32)
    logz = _compute_logz(table)
    logz_pad = jnp.pad(logz[:, 0], (0, LOGZ_PAD - VOCAB))
    tslices = [table[:, k * KW:(k + 1) * KW] for k in range(KCH)]
    q, partials = _sc_main(*tslices, idxT, tgtT, logz_pad)
    logits = jnp.transpose(q, (2, 4, 0, 1, 3)).reshape(BATCH, SEQ, VOCAB)
    loss = _finalize_loss(partials)[0, 0]
    return logits, loss
